# baseline (speedup 1.0000x reference)
import functools

import jax
import jax.numpy as jnp
from jax import lax
from jax.experimental import pallas as pl
from jax.experimental.pallas import tpu as pltpu
from jax.experimental.pallas import tpu_sc as plsc

B = 4
L = 8192
C = 768
K = 8
LANES = 16
NCG = C // LANES
NTASK = B * NCG
NW = 32
TPW = NTASK // NW
LC = 1024
NCHUNK = L // LC


def _insert_rows(V, rows):
    for v in rows:
        out = []
        for kk in range(K):
            hi = jnp.maximum(V[kk], v)
            v = jnp.minimum(V[kk], v)
            out.append(hi)
        V = tuple(out)
    return V


G = 8


def _inner(buf, V):
    def group_body(i, V):
        base = i * G
        rows = [buf[base + j] for j in range(G)]
        m = rows[0]
        for r in rows[1:]:
            m = jnp.maximum(m, r)
        anyb = jnp.any(m > V[K - 1])
        return lax.cond(anyb, lambda V: _insert_rows(V, rows), lambda V: V, V)

    return lax.fori_loop(0, LC // G, group_body, V)


@functools.partial(
    pl.kernel,
    mesh=plsc.VectorSubcoreMesh(core_axis_name="c", subcore_axis_name="s"),
    out_type=jax.ShapeDtypeStruct((B, K, C), jnp.float32),
    scratch_types=[
        pltpu.VMEM((LC, LANES), jnp.float32),
        pltpu.VMEM((LC, LANES), jnp.float32),
        pltpu.VMEM((K, LANES), jnp.float32),
        pltpu.SemaphoreType.DMA,
        pltpu.SemaphoreType.DMA,
    ],
    compiler_params=pltpu.CompilerParams(
        use_tc_tiling_on_sc=False, needs_layout_passes=False
    ),
)
def _topk_sc(x_hbm, out_hbm, buf0, buf1, outb, sem0, sem1):
    wid = lax.axis_index("s") * 2 + lax.axis_index("c")
    bufs = (buf0, buf1)
    sems = (sem0, sem1)

    for t in range(TPW):
        g = t * NW + wid
        b = g // NCG
        cg = g - b * NCG
        c0 = cg * LANES

        def src(chunk, b=b, c0=c0):
            return x_hbm.at[b, pl.ds(chunk * LC, LC), pl.ds(c0, LANES)]

        def start(chunk, slot):
            pltpu.async_copy(src(chunk), bufs[slot], sems[slot])

        def wait(chunk, slot):
            pltpu.make_async_copy(src(chunk), bufs[slot], sems[slot]).wait()

        start(0, 0)
        neg_inf = jnp.full((LANES,), -jnp.inf, dtype=jnp.float32)
        V = tuple(neg_inf for _ in range(K))

        def pair_body(p, V):
            c = 2 * p
            start(c + 1, 1)
            wait(c, 0)
            V = _inner(buf0, V)

            @pl.when(p < NCHUNK // 2 - 1)
            def _():
                start(c + 2, 0)

            wait(c + 1, 1)
            V = _inner(buf1, V)
            return V

        V = lax.fori_loop(0, NCHUNK // 2, pair_body, V)

        for kk in range(K):
            outb[kk] = V[kk]
        pltpu.sync_copy(outb, out_hbm.at[b, pl.ds(0, K), pl.ds(c0, LANES)])


def kernel(inputs):
    return _topk_sc(inputs)


# speedup vs baseline: 1.6680x; 1.6680x over previous
import functools

import jax
import jax.numpy as jnp
from jax import lax
from jax.experimental import pallas as pl
from jax.experimental.pallas import tpu as pltpu
from jax.experimental.pallas import tpu_sc as plsc

B = 4
L = 8192
C = 768
K = 8
LANES = 16
NCG = C // LANES
NTASK = B * NCG
NW = 32
TPW = NTASK // NW
LC = 1024
NCHUNK = L // LC


G = 8

_SORT8 = (
    (0, 1), (2, 3), (4, 5), (6, 7),
    (0, 2), (1, 3), (4, 6), (5, 7),
    (1, 2), (5, 6),
    (0, 4), (1, 5), (2, 6), (3, 7),
    (2, 4), (3, 5),
    (1, 2), (3, 4), (5, 6),
)
_BMERGE = (
    (0, 4), (1, 5), (2, 6), (3, 7),
    (0, 2), (1, 3), (4, 6), (5, 7),
    (0, 1), (2, 3), (4, 5), (6, 7),
)


def _inner(buf, V):
    def group_body(i, V):
        base = i * G
        rows = [buf[base + j] for j in range(G)]
        for a, b in _SORT8:
            hi = jnp.maximum(rows[a], rows[b])
            lo = jnp.minimum(rows[a], rows[b])
            rows[a], rows[b] = hi, lo
        M = [jnp.maximum(V[k], rows[K - 1 - k]) for k in range(K)]
        for a, b in _BMERGE:
            hi = jnp.maximum(M[a], M[b])
            lo = jnp.minimum(M[a], M[b])
            M[a], M[b] = hi, lo
        return tuple(M)

    return lax.fori_loop(0, LC // G, group_body, V)


@functools.partial(
    pl.kernel,
    mesh=plsc.VectorSubcoreMesh(core_axis_name="c", subcore_axis_name="s"),
    out_type=jax.ShapeDtypeStruct((B, K, C), jnp.float32),
    scratch_types=[
        pltpu.VMEM((LC, LANES), jnp.float32),
        pltpu.VMEM((LC, LANES), jnp.float32),
        pltpu.VMEM((K, LANES), jnp.float32),
        pltpu.SemaphoreType.DMA,
        pltpu.SemaphoreType.DMA,
    ],
    compiler_params=pltpu.CompilerParams(
        use_tc_tiling_on_sc=False, needs_layout_passes=False
    ),
)
def _topk_sc(x_hbm, out_hbm, buf0, buf1, outb, sem0, sem1):
    wid = lax.axis_index("s") * 2 + lax.axis_index("c")
    bufs = (buf0, buf1)
    sems = (sem0, sem1)

    for t in range(TPW):
        g = t * NW + wid
        b = g // NCG
        cg = g - b * NCG
        c0 = cg * LANES

        def src(chunk, b=b, c0=c0):
            return x_hbm.at[b, pl.ds(chunk * LC, LC), pl.ds(c0, LANES)]

        def start(chunk, slot):
            pltpu.async_copy(src(chunk), bufs[slot], sems[slot])

        def wait(chunk, slot):
            pltpu.make_async_copy(src(chunk), bufs[slot], sems[slot]).wait()

        start(0, 0)
        neg_inf = jnp.full((LANES,), -jnp.inf, dtype=jnp.float32)
        V = tuple(neg_inf for _ in range(K))

        def pair_body(p, V):
            c = 2 * p
            start(c + 1, 1)
            wait(c, 0)
            V = _inner(buf0, V)

            @pl.when(p < NCHUNK // 2 - 1)
            def _():
                start(c + 2, 0)

            wait(c + 1, 1)
            V = _inner(buf1, V)
            return V

        V = lax.fori_loop(0, NCHUNK // 2, pair_body, V)

        for kk in range(K):
            outb[kk] = V[kk]
        pltpu.sync_copy(outb, out_hbm.at[b, pl.ds(0, K), pl.ds(c0, LANES)])


def kernel(inputs):
    return _topk_sc(inputs)
